# baseline (speedup 1.0000x reference)
import functools

import jax
import jax.numpy as jnp
from jax import lax
from jax.experimental import pallas as pl
from jax.experimental.pallas import tpu as pltpu
from jax.experimental.pallas import tpu_sc as plsc

D = 256
DE = 32
NC, NS, LANES = 2, 16, 16
NW = NC * NS
NJ = D // LANES


def _fold_tables_body(ep, ef, ed, wl, wi, wf, bf, bl, bi, tab_ref, vli_ref):
    wfus = wf[...]
    w_p = wfus[0:32, :]
    w_l = wfus[32:64, :]
    w_f = wfus[64:96, :]
    w_i = wfus[96:128, :]
    w_d = wfus[128:136, :]
    f32 = jnp.float32
    tab_ref[0:256, :] = jnp.dot(ep[...], w_p, preferred_element_type=f32)
    bias_c = (
        bf[...]
        + jnp.dot(bl[...].reshape(1, DE), w_l, preferred_element_type=f32)[0]
        + jnp.dot(bi[...].reshape(1, DE), w_i, preferred_element_type=f32)[0]
    )
    e_f = jnp.dot(ef[...], w_f, preferred_element_type=f32)
    e_d = jnp.dot(ed[...], w_d, preferred_element_type=f32)
    tfd = e_f[None, :, :] + e_d[:, None, :] + bias_c[None, None, :]
    tab_ref[256:384, :] = tfd.reshape(128, D)
    v_l = jnp.dot(wl[...], w_l, preferred_element_type=f32)
    v_i = jnp.dot(wi[...], w_i, preferred_element_type=f32)
    vli_ref[...] = jnp.concatenate([v_l, v_i], axis=0)


def _fold_tables(ep, ef, ed, wl, wi, wf, bf, bl, bi):
    return pl.pallas_call(
        _fold_tables_body,
        out_shape=[
            jax.ShapeDtypeStruct((384, D), jnp.float32),
            jax.ShapeDtypeStruct((2, D), jnp.float32),
        ],
    )(ep, ef, ed, wl, wi, wf, bf, bl, bi)


def _rsqrt_nr(a):
    half = a * 0.5
    i = plsc.bitcast(a, jnp.int32)
    i = 0x5F3759DF - lax.shift_right_arithmetic(i, 1)
    y = plsc.bitcast(i, jnp.float32)
    for _ in range(3):
        y = y * (1.5 - half * y * y)
    return y


def _make_sc_kernel(n_rows):
    PW = n_rows // NW
    CH = 1600
    NSTAGE = PW // CH
    CPS = CH // LANES
    NPAIR = (PW // LANES) // 2
    PPS = CPS // 2
    mesh = plsc.VectorSubcoreMesh(core_axis_name="c", subcore_axis_name="s")

    @functools.partial(
        pl.kernel,
        out_type=jax.ShapeDtypeStruct((n_rows * D,), jnp.float32),
        mesh=mesh,
        compiler_params=pltpu.CompilerParams(
            use_tc_tiling_on_sc=False, needs_layout_passes=False),
        scratch_types=[
            pltpu.VMEM((512,), jnp.float32),
            pltpu.VMEM((512,), jnp.float32),
            pltpu.VMEM((CH,), jnp.float32),
            pltpu.VMEM((CH,), jnp.float32),
            pltpu.VMEM((CH,), jnp.float32),
            pltpu.VMEM((CH,), jnp.float32),
            pltpu.VMEM((CH,), jnp.float32),
            pltpu.VMEM((LANES, D), jnp.float32),
            pltpu.VMEM((LANES, D), jnp.float32),
            pltpu.VMEM((LANES, D), jnp.float32),
            pltpu.VMEM((LANES, D), jnp.float32),
            pltpu.VMEM((LANES * D,), jnp.float32),
            pltpu.VMEM((LANES * D,), jnp.float32),
            pltpu.SemaphoreType.DMA,
            pltpu.SemaphoreType.DMA,
            pltpu.SemaphoreType.DMA,
            pltpu.SemaphoreType.DMA,
        ],
    )
    def sc_kernel(tab_h, vli_h, gb_h, xp_h, xl_h, xf_h, xi_h, xd_h, out_h,
                  vliv, gbv, bufp, bufl, buff, bufi, bufd,
                  pA, fA, pB, fB, oA, oB, sgA, sgB, soA, soB):
        wid = lax.axis_index("s") * NC + lax.axis_index("c")
        base = wid * PW
        pltpu.sync_copy(vli_h, vliv)
        pltpu.sync_copy(gb_h, gbv)

        zv = jnp.zeros((LANES,), jnp.int32)
        fz = jnp.zeros((LANES,), jnp.float32)
        inv_d = jnp.float32(1.0 / D)

        def restage(stage):
            cb = base + stage * CH
            pltpu.sync_copy(xp_h.at[pl.ds(cb, CH)], bufp)
            pltpu.sync_copy(xl_h.at[pl.ds(cb, CH)], bufl)
            pltpu.sync_copy(xf_h.at[pl.ds(cb, CH)], buff)
            pltpu.sync_copy(xi_h.at[pl.ds(cb, CH)], bufi)
            pltpu.sync_copy(xd_h.at[pl.ds(cb, CH)], bufd)

        def issue_gather(loc, pbuf, fbuf, sem):
            p16 = jnp.clip(bufp[pl.ds(loc, LANES)].astype(jnp.int32), 0, 255)
            f16 = jnp.clip(buff[pl.ds(loc, LANES)].astype(jnp.int32), 0, 63)
            d16 = jnp.clip(bufd[pl.ds(loc, LANES)].astype(jnp.int32), 0, 1)
            fd16 = 256 + d16 * 64 + f16
            pltpu.async_copy(tab_h.at[p16], pbuf, sem)
            pltpu.async_copy(tab_h.at[fd16], fbuf, sem)

        def wait_gather(pbuf, fbuf, sem):
            pltpu.make_async_copy(tab_h.at[zv], pbuf, sem).wait()
            pltpu.make_async_copy(tab_h.at[zv], fbuf, sem).wait()

        def wait_out(obuf, sem):
            pltpu.make_async_copy(obuf, out_h.at[pl.ds(0, LANES * D)],
                                  sem).wait()

        vls = [vliv[pl.ds(j * LANES, LANES)] for j in range(NJ)]
        vis = [vliv[pl.ds(D + j * LANES, LANES)] for j in range(NJ)]
        gms = [gbv[pl.ds(j * LANES, LANES)] for j in range(NJ)]
        bts = [gbv[pl.ds(D + j * LANES, LANES)] for j in range(NJ)]

        def compute(loc, pbuf, fbuf, obuf):
            @plsc.parallel_loop(0, LANES)
            def _(r):
                lenv = plsc.load_gather(bufl, [zv + (loc + r)])
                iatv = plsc.load_gather(bufi, [zv + (loc + r)])
                hjs = []
                s = fz
                q = fz
                for j in range(NJ):
                    hj = (pbuf[r, pl.ds(j * LANES, LANES)]
                          + fbuf[r, pl.ds(j * LANES, LANES)]
                          + lenv * vls[j] + iatv * vis[j])
                    hjs.append(hj)
                    s = s + hj
                    q = q + hj * hj
                st = jnp.sum(s)
                qt = jnp.sum(q)
                m = st * inv_d
                var = qt * inv_d - m * m
                rv = _rsqrt_nr(fz + (var + 1e-5))
                mv = fz + m
                ro = r * D
                for j in range(NJ):
                    obuf[pl.ds(ro + j * LANES, LANES)] = (
                        (hjs[j] - mv) * rv * gms[j] + bts[j])

        def pair_body(k2, carry):
            a = 2 * k2
            b = a + 1
            loc_a = lax.rem(a, CPS) * LANES
            loc_b = loc_a + LANES

            @pl.when(lax.rem(k2, PPS) == 0)
            def _w0():
                restage(lax.div(k2, PPS))
                issue_gather(loc_a, pA, fA, sgA)

            issue_gather(loc_b, pB, fB, sgB)

            @pl.when(k2 >= 1)
            def _w1():
                wait_out(oA, soA)

            wait_gather(pA, fA, sgA)
            compute(loc_a, pA, fA, oA)
            pltpu.async_copy(oA, out_h.at[pl.ds((base + a * LANES) * D,
                                                LANES * D)], soA)

            @pl.when(lax.rem(k2, PPS) != PPS - 1)
            def _w2():
                issue_gather(loc_b + LANES, pA, fA, sgA)

            @pl.when(k2 >= 1)
            def _w3():
                wait_out(oB, soB)

            wait_gather(pB, fB, sgB)
            compute(loc_b, pB, fB, oB)
            pltpu.async_copy(oB, out_h.at[pl.ds((base + b * LANES) * D,
                                                LANES * D)], soB)
            return carry

        lax.fori_loop(0, NPAIR, pair_body, 0)
        wait_out(oA, soA)
        wait_out(oB, soB)

    return sc_kernel


def kernel(x, emb_proto, emb_flags, emb_dir, w_len, b_len, w_iat, b_iat,
           w_fus, b_fus, gamma, beta):
    B, L, _ = x.shape
    n_rows = B * L
    tab, vli = _fold_tables(emb_proto, emb_flags, emb_dir, w_len, w_iat,
                            w_fus, b_fus, b_len, b_iat)
    gb = jnp.concatenate([gamma, beta]).reshape(512)
    vli_flat = vli.reshape(512)
    xf = x.reshape(n_rows, 5)
    cols = [xf[:, k] for k in range(5)]
    out = _make_sc_kernel(n_rows)(tab, vli_flat, gb, *cols)
    return out.reshape(B, L, D)


# speedup vs baseline: 1.0024x; 1.0024x over previous
import functools

import jax
import jax.numpy as jnp
from jax import lax
from jax.experimental import pallas as pl
from jax.experimental.pallas import tpu as pltpu
from jax.experimental.pallas import tpu_sc as plsc

D = 256
DE = 32
NC, NS, LANES = 2, 16, 16
NW = NC * NS
NJ = D // LANES


def _fold_tables_body(ep, ef, ed, wl, wi, wf, bf, bl, bi, tab_ref, vli_ref):
    wfus = wf[...]
    w_p = wfus[0:32, :]
    w_l = wfus[32:64, :]
    w_f = wfus[64:96, :]
    w_i = wfus[96:128, :]
    w_d = wfus[128:136, :]
    f32 = jnp.float32
    tab_ref[0:256, :] = jnp.dot(ep[...], w_p, preferred_element_type=f32)
    bias_c = (
        bf[...]
        + jnp.dot(bl[...].reshape(1, DE), w_l, preferred_element_type=f32)[0]
        + jnp.dot(bi[...].reshape(1, DE), w_i, preferred_element_type=f32)[0]
    )
    e_f = jnp.dot(ef[...], w_f, preferred_element_type=f32)
    e_d = jnp.dot(ed[...], w_d, preferred_element_type=f32)
    tfd = e_f[None, :, :] + e_d[:, None, :] + bias_c[None, None, :]
    tab_ref[256:384, :] = tfd.reshape(128, D)
    v_l = jnp.dot(wl[...], w_l, preferred_element_type=f32)
    v_i = jnp.dot(wi[...], w_i, preferred_element_type=f32)
    vli_ref[...] = jnp.concatenate([v_l, v_i], axis=0)


def _fold_tables(ep, ef, ed, wl, wi, wf, bf, bl, bi):
    return pl.pallas_call(
        _fold_tables_body,
        out_shape=[
            jax.ShapeDtypeStruct((384, D), jnp.float32),
            jax.ShapeDtypeStruct((2, D), jnp.float32),
        ],
    )(ep, ef, ed, wl, wi, wf, bf, bl, bi)


def _rsqrt_nr(a):
    half = a * 0.5
    i = plsc.bitcast(a, jnp.int32)
    i = 0x5F3759DF - lax.shift_right_arithmetic(i, 1)
    y = plsc.bitcast(i, jnp.float32)
    for _ in range(3):
        y = y * (1.5 - half * y * y)
    return y


def _make_sc_kernel(n_rows):
    PW = n_rows // NW
    R = 64
    CH = 3200
    NSTAGE = PW // CH
    CPS = CH // R
    NPAIR = (PW // R) // 2
    PPS = CPS // 2
    mesh = plsc.VectorSubcoreMesh(core_axis_name="c", subcore_axis_name="s")

    @functools.partial(
        pl.kernel,
        out_type=jax.ShapeDtypeStruct((n_rows * D,), jnp.float32),
        mesh=mesh,
        compiler_params=pltpu.CompilerParams(
            use_tc_tiling_on_sc=False, needs_layout_passes=False),
        scratch_types=[
            pltpu.VMEM((512,), jnp.float32),
            pltpu.VMEM((512,), jnp.float32),
            pltpu.VMEM((CH,), jnp.float32),
            pltpu.VMEM((CH,), jnp.float32),
            pltpu.VMEM((CH,), jnp.float32),
            pltpu.VMEM((CH,), jnp.float32),
            pltpu.VMEM((CH,), jnp.float32),
            pltpu.VMEM((R, D), jnp.float32),
            pltpu.VMEM((R, D), jnp.float32),
            pltpu.VMEM((R, D), jnp.float32),
            pltpu.VMEM((R, D), jnp.float32),
            pltpu.VMEM((R * D,), jnp.float32),
            pltpu.VMEM((R * D,), jnp.float32),
            pltpu.VMEM((R,), jnp.int32),
            pltpu.VMEM((R,), jnp.int32),
            pltpu.VMEM((R,), jnp.int32),
            pltpu.VMEM((R,), jnp.int32),
            pltpu.SemaphoreType.DMA,
            pltpu.SemaphoreType.DMA,
            pltpu.SemaphoreType.DMA,
            pltpu.SemaphoreType.DMA,
        ],
    )
    def sc_kernel(tab_h, vli_h, gb_h, xp_h, xl_h, xf_h, xi_h, xd_h, out_h,
                  vliv, gbv, bufp, bufl, buff, bufi, bufd,
                  pA, fA, pB, fB, oA, oB, ipA, ifA, ipB, ifB,
                  sgA, sgB, soA, soB):
        wid = lax.axis_index("s") * NC + lax.axis_index("c")
        base = wid * PW
        pltpu.sync_copy(vli_h, vliv)
        pltpu.sync_copy(gb_h, gbv)

        zv = jnp.zeros((LANES,), jnp.int32)
        fz = jnp.zeros((LANES,), jnp.float32)
        inv_d = jnp.float32(1.0 / D)

        def restage(stage):
            cb = base + stage * CH
            pltpu.sync_copy(xp_h.at[pl.ds(cb, CH)], bufp)
            pltpu.sync_copy(xl_h.at[pl.ds(cb, CH)], bufl)
            pltpu.sync_copy(xf_h.at[pl.ds(cb, CH)], buff)
            pltpu.sync_copy(xi_h.at[pl.ds(cb, CH)], bufi)
            pltpu.sync_copy(xd_h.at[pl.ds(cb, CH)], bufd)

        def issue_gather(loc, ipx, ifx, pbuf, fbuf, sem):
            for i in range(R // LANES):
                o = i * LANES
                p16 = jnp.clip(
                    bufp[pl.ds(loc + o, LANES)].astype(jnp.int32), 0, 255)
                f16 = jnp.clip(
                    buff[pl.ds(loc + o, LANES)].astype(jnp.int32), 0, 63)
                d16 = jnp.clip(
                    bufd[pl.ds(loc + o, LANES)].astype(jnp.int32), 0, 1)
                ipx[pl.ds(o, LANES)] = p16
                ifx[pl.ds(o, LANES)] = 256 + d16 * 64 + f16
            pltpu.async_copy(tab_h.at[ipx], pbuf, sem)
            pltpu.async_copy(tab_h.at[ifx], fbuf, sem)

        def wait_gather(ipx, ifx, pbuf, fbuf, sem):
            pltpu.make_async_copy(tab_h.at[ipx], pbuf, sem).wait()
            pltpu.make_async_copy(tab_h.at[ifx], fbuf, sem).wait()

        def wait_out(obuf, sem):
            pltpu.make_async_copy(obuf, out_h.at[pl.ds(0, R * D)],
                                  sem).wait()

        vls = [vliv[pl.ds(j * LANES, LANES)] for j in range(NJ)]
        vis = [vliv[pl.ds(D + j * LANES, LANES)] for j in range(NJ)]
        gms = [gbv[pl.ds(j * LANES, LANES)] for j in range(NJ)]
        bts = [gbv[pl.ds(D + j * LANES, LANES)] for j in range(NJ)]

        def compute(loc, pbuf, fbuf, obuf):
            @plsc.parallel_loop(0, R)
            def _(r):
                lenv = plsc.load_gather(bufl, [zv + (loc + r)])
                iatv = plsc.load_gather(bufi, [zv + (loc + r)])
                hjs = []
                s = fz
                q = fz
                for j in range(NJ):
                    hj = (pbuf[r, pl.ds(j * LANES, LANES)]
                          + fbuf[r, pl.ds(j * LANES, LANES)]
                          + lenv * vls[j] + iatv * vis[j])
                    hjs.append(hj)
                    s = s + hj
                    q = q + hj * hj
                st = jnp.sum(s)
                qt = jnp.sum(q)
                m = st * inv_d
                var = qt * inv_d - m * m
                rv = _rsqrt_nr(fz + (var + 1e-5))
                mv = fz + m
                ro = r * D
                for j in range(NJ):
                    obuf[pl.ds(ro + j * LANES, LANES)] = (
                        (hjs[j] - mv) * rv * gms[j] + bts[j])

        def pair_body(k2, carry):
            a = 2 * k2
            b = a + 1
            loc_a = lax.rem(a, CPS) * R
            loc_b = loc_a + R

            @pl.when(lax.rem(k2, PPS) == 0)
            def _w0():
                restage(lax.div(k2, PPS))
                issue_gather(loc_a, ipA, ifA, pA, fA, sgA)

            issue_gather(loc_b, ipB, ifB, pB, fB, sgB)

            @pl.when(k2 >= 1)
            def _w1():
                wait_out(oA, soA)

            wait_gather(ipA, ifA, pA, fA, sgA)
            compute(loc_a, pA, fA, oA)
            pltpu.async_copy(oA, out_h.at[pl.ds((base + a * R) * D,
                                                R * D)], soA)

            @pl.when(lax.rem(k2, PPS) != PPS - 1)
            def _w2():
                issue_gather(loc_b + R, ipA, ifA, pA, fA, sgA)

            @pl.when(k2 >= 1)
            def _w3():
                wait_out(oB, soB)

            wait_gather(ipB, ifB, pB, fB, sgB)
            compute(loc_b, pB, fB, oB)
            pltpu.async_copy(oB, out_h.at[pl.ds((base + b * R) * D,
                                                R * D)], soB)
            return carry

        lax.fori_loop(0, NPAIR, pair_body, 0)
        wait_out(oA, soA)
        wait_out(oB, soB)

    return sc_kernel


def kernel(x, emb_proto, emb_flags, emb_dir, w_len, b_len, w_iat, b_iat,
           w_fus, b_fus, gamma, beta):
    B, L, _ = x.shape
    n_rows = B * L
    tab, vli = _fold_tables(emb_proto, emb_flags, emb_dir, w_len, w_iat,
                            w_fus, b_fus, b_len, b_iat)
    gb = jnp.concatenate([gamma, beta]).reshape(512)
    vli_flat = vli.reshape(512)
    xf = x.reshape(n_rows, 5)
    cols = [xf[:, k] for k in range(5)]
    out = _make_sc_kernel(n_rows)(tab, vli_flat, gb, *cols)
    return out.reshape(B, L, D)


# speedup vs baseline: 7.6461x; 7.6282x over previous
import functools

import jax
import jax.numpy as jnp
from jax import lax
from jax.experimental import pallas as pl
from jax.experimental.pallas import tpu as pltpu
from jax.experimental.pallas import tpu_sc as plsc

D = 256
DE = 32
NC, NS, LANES = 2, 16, 16
NW = NC * NS
NJ = D // LANES


def _fold_tables_body(ep, ef, ed, wl, wi, wf, bf, bl, bi, tab_ref, vli_ref):
    wfus = wf[...]
    w_p = wfus[0:32, :]
    w_l = wfus[32:64, :]
    w_f = wfus[64:96, :]
    w_i = wfus[96:128, :]
    w_d = wfus[128:136, :]
    f32 = jnp.float32
    tab_ref[0:256, :] = jnp.dot(ep[...], w_p, preferred_element_type=f32)
    bias_c = (
        bf[...]
        + jnp.dot(bl[...].reshape(1, DE), w_l, preferred_element_type=f32)[0]
        + jnp.dot(bi[...].reshape(1, DE), w_i, preferred_element_type=f32)[0]
    )
    e_f = jnp.dot(ef[...], w_f, preferred_element_type=f32)
    e_d = jnp.dot(ed[...], w_d, preferred_element_type=f32)
    tfd = e_f[None, :, :] + e_d[:, None, :] + bias_c[None, None, :]
    tab_ref[256:384, :] = tfd.reshape(128, D)
    v_l = jnp.dot(wl[...], w_l, preferred_element_type=f32)
    v_i = jnp.dot(wi[...], w_i, preferred_element_type=f32)
    vli_ref[...] = jnp.concatenate([v_l, v_i], axis=0)


def _fold_tables(ep, ef, ed, wl, wi, wf, bf, bl, bi):
    return pl.pallas_call(
        _fold_tables_body,
        out_shape=[
            jax.ShapeDtypeStruct((384, D), jnp.float32),
            jax.ShapeDtypeStruct((2, D), jnp.float32),
        ],
    )(ep, ef, ed, wl, wi, wf, bf, bl, bi)


def _rsqrt_nr(a):
    half = a * 0.5
    i = plsc.bitcast(a, jnp.int32)
    i = 0x5F3759DF - lax.shift_right_arithmetic(i, 1)
    y = plsc.bitcast(i, jnp.float32)
    for _ in range(3):
        y = y * (1.5 - half * y * y)
    return y


def _make_sc_kernel(n_rows):
    PW = n_rows // NW
    R = 64
    CH = 3200
    NSTAGE = PW // CH
    CPS = CH // R
    NPAIR = (PW // R) // 2
    PPS = CPS // 2
    mesh = plsc.VectorSubcoreMesh(core_axis_name="c", subcore_axis_name="s")

    @functools.partial(
        pl.kernel,
        out_type=jax.ShapeDtypeStruct((n_rows * D,), jnp.float32),
        mesh=mesh,
        compiler_params=pltpu.CompilerParams(
            use_tc_tiling_on_sc=False, needs_layout_passes=False),
        scratch_types=[
            pltpu.VMEM_SHARED((384, D), jnp.float32),
            pltpu.VMEM((512,), jnp.float32),
            pltpu.VMEM((512,), jnp.float32),
            pltpu.VMEM((CH,), jnp.float32),
            pltpu.VMEM((CH,), jnp.float32),
            pltpu.VMEM((CH,), jnp.float32),
            pltpu.VMEM((CH,), jnp.float32),
            pltpu.VMEM((CH,), jnp.float32),
            pltpu.VMEM((R, D), jnp.float32),
            pltpu.VMEM((R, D), jnp.float32),
            pltpu.VMEM((R, D), jnp.float32),
            pltpu.VMEM((R, D), jnp.float32),
            pltpu.VMEM((R * D,), jnp.float32),
            pltpu.VMEM((R * D,), jnp.float32),
            pltpu.VMEM((R,), jnp.int32),
            pltpu.VMEM((R,), jnp.int32),
            pltpu.VMEM((R,), jnp.int32),
            pltpu.VMEM((R,), jnp.int32),
            pltpu.SemaphoreType.DMA,
            pltpu.SemaphoreType.DMA,
            pltpu.SemaphoreType.DMA,
            pltpu.SemaphoreType.DMA,
        ],
    )
    def sc_kernel(tab_h, vli_h, gb_h, xp_h, xl_h, xf_h, xi_h, xd_h, out_h,
                  spT, vliv, gbv, bufp, bufl, buff, bufi, bufd,
                  pA, fA, pB, fB, oA, oB, ipA, ifA, ipB, ifB,
                  sgA, sgB, soA, soB):
        sid = lax.axis_index("s")
        wid = sid * NC + lax.axis_index("c")
        base = wid * PW

        @pl.when(sid == 0)
        def _stage_tables():
            pltpu.sync_copy(tab_h, spT)

        pltpu.sync_copy(vli_h, vliv)
        pltpu.sync_copy(gb_h, gbv)
        plsc.subcore_barrier()

        zv = jnp.zeros((LANES,), jnp.int32)
        fz = jnp.zeros((LANES,), jnp.float32)
        inv_d = jnp.float32(1.0 / D)

        def restage(stage):
            cb = base + stage * CH
            pltpu.sync_copy(xp_h.at[pl.ds(cb, CH)], bufp)
            pltpu.sync_copy(xl_h.at[pl.ds(cb, CH)], bufl)
            pltpu.sync_copy(xf_h.at[pl.ds(cb, CH)], buff)
            pltpu.sync_copy(xi_h.at[pl.ds(cb, CH)], bufi)
            pltpu.sync_copy(xd_h.at[pl.ds(cb, CH)], bufd)

        def issue_gather(loc, ipx, ifx, pbuf, fbuf, sem):
            for i in range(R // LANES):
                o = i * LANES
                p16 = jnp.clip(
                    bufp[pl.ds(loc + o, LANES)].astype(jnp.int32), 0, 255)
                f16 = jnp.clip(
                    buff[pl.ds(loc + o, LANES)].astype(jnp.int32), 0, 63)
                d16 = jnp.clip(
                    bufd[pl.ds(loc + o, LANES)].astype(jnp.int32), 0, 1)
                ipx[pl.ds(o, LANES)] = p16
                ifx[pl.ds(o, LANES)] = 256 + d16 * 64 + f16
            pltpu.async_copy(spT.at[ipx], pbuf, sem)
            pltpu.async_copy(spT.at[ifx], fbuf, sem)

        def wait_gather(ipx, ifx, pbuf, fbuf, sem):
            pltpu.make_async_copy(spT.at[ipx], pbuf, sem).wait()
            pltpu.make_async_copy(spT.at[ifx], fbuf, sem).wait()

        def wait_out(obuf, sem):
            pltpu.make_async_copy(obuf, out_h.at[pl.ds(0, R * D)],
                                  sem).wait()

        vls = [vliv[pl.ds(j * LANES, LANES)] for j in range(NJ)]
        vis = [vliv[pl.ds(D + j * LANES, LANES)] for j in range(NJ)]
        gms = [gbv[pl.ds(j * LANES, LANES)] for j in range(NJ)]
        bts = [gbv[pl.ds(D + j * LANES, LANES)] for j in range(NJ)]

        def compute(loc, pbuf, fbuf, obuf):
            @plsc.parallel_loop(0, R)
            def _(r):
                lenv = plsc.load_gather(bufl, [zv + (loc + r)])
                iatv = plsc.load_gather(bufi, [zv + (loc + r)])
                hjs = []
                s = fz
                q = fz
                for j in range(NJ):
                    hj = (pbuf[r, pl.ds(j * LANES, LANES)]
                          + fbuf[r, pl.ds(j * LANES, LANES)]
                          + lenv * vls[j] + iatv * vis[j])
                    hjs.append(hj)
                    s = s + hj
                    q = q + hj * hj
                st = jnp.sum(s)
                qt = jnp.sum(q)
                m = st * inv_d
                var = qt * inv_d - m * m
                rv = _rsqrt_nr(fz + (var + 1e-5))
                mv = fz + m
                ro = r * D
                for j in range(NJ):
                    obuf[pl.ds(ro + j * LANES, LANES)] = (
                        (hjs[j] - mv) * rv * gms[j] + bts[j])

        def pair_body(k2, carry):
            a = 2 * k2
            b = a + 1
            loc_a = lax.rem(a, CPS) * R
            loc_b = loc_a + R

            @pl.when(lax.rem(k2, PPS) == 0)
            def _w0():
                restage(lax.div(k2, PPS))
                issue_gather(loc_a, ipA, ifA, pA, fA, sgA)

            issue_gather(loc_b, ipB, ifB, pB, fB, sgB)

            @pl.when(k2 >= 1)
            def _w1():
                wait_out(oA, soA)

            wait_gather(ipA, ifA, pA, fA, sgA)
            compute(loc_a, pA, fA, oA)
            pltpu.async_copy(oA, out_h.at[pl.ds((base + a * R) * D,
                                                R * D)], soA)

            @pl.when(lax.rem(k2, PPS) != PPS - 1)
            def _w2():
                issue_gather(loc_b + R, ipA, ifA, pA, fA, sgA)

            @pl.when(k2 >= 1)
            def _w3():
                wait_out(oB, soB)

            wait_gather(ipB, ifB, pB, fB, sgB)
            compute(loc_b, pB, fB, oB)
            pltpu.async_copy(oB, out_h.at[pl.ds((base + b * R) * D,
                                                R * D)], soB)
            return carry

        lax.fori_loop(0, NPAIR, pair_body, 0)
        wait_out(oA, soA)
        wait_out(oB, soB)

    return sc_kernel


def kernel(x, emb_proto, emb_flags, emb_dir, w_len, b_len, w_iat, b_iat,
           w_fus, b_fus, gamma, beta):
    B, L, _ = x.shape
    n_rows = B * L
    tab, vli = _fold_tables(emb_proto, emb_flags, emb_dir, w_len, w_iat,
                            w_fus, b_fus, b_len, b_iat)
    gb = jnp.concatenate([gamma, beta]).reshape(512)
    vli_flat = vli.reshape(512)
    xf = x.reshape(n_rows, 5)
    cols = [xf[:, k] for k in range(5)]
    out = _make_sc_kernel(n_rows)(tab, vli_flat, gb, *cols)
    return out.reshape(B, L, D)


# speedup vs baseline: 9.6123x; 1.2571x over previous
import functools

import jax
import jax.numpy as jnp
from jax import lax
from jax.experimental import pallas as pl
from jax.experimental.pallas import tpu as pltpu
from jax.experimental.pallas import tpu_sc as plsc

D = 256
DE = 32
NC, NS, LANES = 2, 16, 16
NW = NC * NS
NJ = D // LANES


def _fold_tables_body(ep, ef, ed, wl, wi, wf, bf, bl, bi, tab_ref, vli_ref):
    wfus = wf[...]
    w_p = wfus[0:32, :]
    w_l = wfus[32:64, :]
    w_f = wfus[64:96, :]
    w_i = wfus[96:128, :]
    w_d = wfus[128:136, :]
    f32 = jnp.float32
    tab_ref[0:256, :] = jnp.dot(ep[...], w_p, preferred_element_type=f32)
    bias_c = (
        bf[...]
        + jnp.dot(bl[...].reshape(1, DE), w_l, preferred_element_type=f32)[0]
        + jnp.dot(bi[...].reshape(1, DE), w_i, preferred_element_type=f32)[0]
    )
    e_f = jnp.dot(ef[...], w_f, preferred_element_type=f32)
    e_d = jnp.dot(ed[...], w_d, preferred_element_type=f32)
    tfd = e_f[None, :, :] + e_d[:, None, :] + bias_c[None, None, :]
    tab_ref[256:384, :] = tfd.reshape(128, D)
    v_l = jnp.dot(wl[...], w_l, preferred_element_type=f32)
    v_i = jnp.dot(wi[...], w_i, preferred_element_type=f32)
    vli_ref[...] = jnp.concatenate([v_l, v_i], axis=0)


def _fold_tables(ep, ef, ed, wl, wi, wf, bf, bl, bi):
    return pl.pallas_call(
        _fold_tables_body,
        out_shape=[
            jax.ShapeDtypeStruct((384, D), jnp.float32),
            jax.ShapeDtypeStruct((2, D), jnp.float32),
        ],
    )(ep, ef, ed, wl, wi, wf, bf, bl, bi)


def _rsqrt_nr(a):
    half = a * 0.5
    i = plsc.bitcast(a, jnp.int32)
    i = 0x5F3759DF - lax.shift_right_arithmetic(i, 1)
    y = plsc.bitcast(i, jnp.float32)
    for _ in range(3):
        y = y * (1.5 - half * y * y)
    return y


def _make_sc_kernel(n_rows):
    PW = n_rows // NW
    R = 64
    CH = 3200
    NSTAGE = PW // CH
    CPS = CH // R
    NPAIR = (PW // R) // 2
    PPS = CPS // 2
    mesh = plsc.VectorSubcoreMesh(core_axis_name="c", subcore_axis_name="s")

    @functools.partial(
        pl.kernel,
        out_type=jax.ShapeDtypeStruct((n_rows * D,), jnp.float32),
        mesh=mesh,
        compiler_params=pltpu.CompilerParams(
            use_tc_tiling_on_sc=False, needs_layout_passes=False),
        scratch_types=[
            pltpu.VMEM_SHARED((384, D), jnp.float32),
            pltpu.VMEM((512,), jnp.float32),
            pltpu.VMEM((512,), jnp.float32),
            pltpu.VMEM((CH,), jnp.float32),
            pltpu.VMEM((CH,), jnp.float32),
            pltpu.VMEM((CH,), jnp.float32),
            pltpu.VMEM((CH,), jnp.float32),
            pltpu.VMEM((CH,), jnp.float32),
            pltpu.VMEM((R, D), jnp.float32),
            pltpu.VMEM((R, D), jnp.float32),
            pltpu.VMEM((R, D), jnp.float32),
            pltpu.VMEM((R, D), jnp.float32),
            pltpu.VMEM((R * D,), jnp.float32),
            pltpu.VMEM((R * D,), jnp.float32),
            pltpu.VMEM((R,), jnp.int32),
            pltpu.VMEM((R,), jnp.int32),
            pltpu.VMEM((R,), jnp.int32),
            pltpu.VMEM((R,), jnp.int32),
            pltpu.SemaphoreType.DMA,
            pltpu.SemaphoreType.DMA,
            pltpu.SemaphoreType.DMA,
            pltpu.SemaphoreType.DMA,
        ],
    )
    def sc_kernel(tab_h, vli_h, gb_h, xp_h, xl_h, xf_h, xi_h, xd_h, out_h,
                  spT, vliv, gbv, bufp, bufl, buff, bufi, bufd,
                  pA, fA, pB, fB, oA, oB, ipA, ifA, ipB, ifB,
                  sgA, sgB, soA, soB):
        sid = lax.axis_index("s")
        wid = sid * NC + lax.axis_index("c")
        base = wid * PW

        @pl.when(sid == 0)
        def _stage_tables():
            pltpu.sync_copy(tab_h, spT)

        pltpu.sync_copy(vli_h, vliv)
        pltpu.sync_copy(gb_h, gbv)
        plsc.subcore_barrier()

        zv = jnp.zeros((LANES,), jnp.int32)
        fz = jnp.zeros((LANES,), jnp.float32)
        inv_d = jnp.float32(1.0 / D)

        def restage(stage):
            cb = base + stage * CH
            pltpu.sync_copy(xp_h.at[pl.ds(cb, CH)], bufp)
            pltpu.sync_copy(xl_h.at[pl.ds(cb, CH)], bufl)
            pltpu.sync_copy(xf_h.at[pl.ds(cb, CH)], buff)
            pltpu.sync_copy(xi_h.at[pl.ds(cb, CH)], bufi)
            pltpu.sync_copy(xd_h.at[pl.ds(cb, CH)], bufd)

        def issue_gather(loc, ipx, ifx, pbuf, fbuf, sem):
            for i in range(R // LANES):
                o = i * LANES
                p16 = jnp.clip(
                    bufp[pl.ds(loc + o, LANES)].astype(jnp.int32), 0, 255)
                f16 = jnp.clip(
                    buff[pl.ds(loc + o, LANES)].astype(jnp.int32), 0, 63)
                d16 = jnp.clip(
                    bufd[pl.ds(loc + o, LANES)].astype(jnp.int32), 0, 1)
                ipx[pl.ds(o, LANES)] = p16
                ifx[pl.ds(o, LANES)] = 256 + d16 * 64 + f16
            pltpu.async_copy(spT.at[ipx], pbuf, sem)
            pltpu.async_copy(spT.at[ifx], fbuf, sem)

        def wait_gather(ipx, ifx, pbuf, fbuf, sem):
            pltpu.make_async_copy(spT.at[ipx], pbuf, sem).wait()
            pltpu.make_async_copy(spT.at[ifx], fbuf, sem).wait()

        def wait_out(obuf, sem):
            pltpu.make_async_copy(obuf, out_h.at[pl.ds(0, R * D)],
                                  sem).wait()

        vls = [vliv[pl.ds(j * LANES, LANES)] for j in range(NJ)]
        vis = [vliv[pl.ds(D + j * LANES, LANES)] for j in range(NJ)]
        gms = [gbv[pl.ds(j * LANES, LANES)] for j in range(NJ)]
        bts = [gbv[pl.ds(D + j * LANES, LANES)] for j in range(NJ)]

        def compute(loc, pbuf, fbuf, obuf):
            @plsc.parallel_loop(0, R, unroll=4)
            def _(r):
                lenv = plsc.load_gather(bufl, [zv + (loc + r)])
                iatv = plsc.load_gather(bufi, [zv + (loc + r)])
                hjs = []
                s = fz
                q = fz
                for j in range(NJ):
                    hj = (pbuf[r, pl.ds(j * LANES, LANES)]
                          + fbuf[r, pl.ds(j * LANES, LANES)]
                          + lenv * vls[j] + iatv * vis[j])
                    hjs.append(hj)
                    s = s + hj
                    q = q + hj * hj
                st = jnp.sum(s)
                qt = jnp.sum(q)
                m = st * inv_d
                var = qt * inv_d - m * m
                rv = _rsqrt_nr(fz + (var + 1e-5))
                mv = fz + m
                ro = r * D
                for j in range(NJ):
                    obuf[pl.ds(ro + j * LANES, LANES)] = (
                        (hjs[j] - mv) * rv * gms[j] + bts[j])

        def pair_body(k2, carry):
            a = 2 * k2
            b = a + 1
            loc_a = lax.rem(a, CPS) * R
            loc_b = loc_a + R

            @pl.when(lax.rem(k2, PPS) == 0)
            def _w0():
                restage(lax.div(k2, PPS))
                issue_gather(loc_a, ipA, ifA, pA, fA, sgA)

            issue_gather(loc_b, ipB, ifB, pB, fB, sgB)

            @pl.when(k2 >= 1)
            def _w1():
                wait_out(oA, soA)

            wait_gather(ipA, ifA, pA, fA, sgA)
            compute(loc_a, pA, fA, oA)
            pltpu.async_copy(oA, out_h.at[pl.ds((base + a * R) * D,
                                                R * D)], soA)

            @pl.when(lax.rem(k2, PPS) != PPS - 1)
            def _w2():
                issue_gather(loc_b + R, ipA, ifA, pA, fA, sgA)

            @pl.when(k2 >= 1)
            def _w3():
                wait_out(oB, soB)

            wait_gather(ipB, ifB, pB, fB, sgB)
            compute(loc_b, pB, fB, oB)
            pltpu.async_copy(oB, out_h.at[pl.ds((base + b * R) * D,
                                                R * D)], soB)
            return carry

        lax.fori_loop(0, NPAIR, pair_body, 0)
        wait_out(oA, soA)
        wait_out(oB, soB)

    return sc_kernel


def kernel(x, emb_proto, emb_flags, emb_dir, w_len, b_len, w_iat, b_iat,
           w_fus, b_fus, gamma, beta):
    B, L, _ = x.shape
    n_rows = B * L
    tab, vli = _fold_tables(emb_proto, emb_flags, emb_dir, w_len, w_iat,
                            w_fus, b_fus, b_len, b_iat)
    gb = jnp.concatenate([gamma, beta]).reshape(512)
    vli_flat = vli.reshape(512)
    xf = x.reshape(n_rows, 5)
    cols = [xf[:, k] for k in range(5)]
    out = _make_sc_kernel(n_rows)(tab, vli_flat, gb, *cols)
    return out.reshape(B, L, D)
